# baseline (speedup 1.0000x reference)
import functools

import jax
import jax.numpy as jnp
from jax.experimental import pallas as pl


INPUT_DIM = 1024
INTER_DIM = 2048
GATE_NUM = 8
TOP_K = 2

TOK_BLK = 512


def _body(x_ref, ew_ref, gw_ref, ow_ref, o_ref):
    x = x_ref[...]
    xb = x.astype(jnp.bfloat16)
    scores = jax.lax.dot_general(
        xb, gw_ref[...].astype(jnp.bfloat16), (((1,), (1,)), ((), ())),
        preferred_element_type=jnp.float32,
    )
    idx = jax.lax.broadcasted_iota(jnp.int32, scores.shape, 1)
    m1 = jnp.max(scores, axis=1, keepdims=True)
    a1 = jnp.min(jnp.where(scores == m1, idx, GATE_NUM), axis=1, keepdims=True)
    scores2 = jnp.where(idx == a1, -jnp.inf, scores)
    m2 = jnp.max(scores2, axis=1, keepdims=True)
    a2 = jnp.min(jnp.where(scores2 == m2, idx, GATE_NUM), axis=1, keepdims=True)

    xf32 = xb.astype(jnp.float32)
    parts = []
    for e in range(GATE_NUM):
        sel = ((a1 == e) | (a2 == e)).astype(jnp.float32)
        parts.append((xf32 * (m1 * sel)).astype(jnp.bfloat16))
    xcat = jnp.concatenate(parts, axis=1)
    acc = jax.lax.dot_general(
        xcat, ew_ref[...], (((1,), (1,)), ((), ())),
        preferred_element_type=jnp.float32,
    )
    g = jnp.square(jnp.maximum(acc, 0.0)).astype(jnp.bfloat16)
    o_ref[...] = jax.lax.dot_general(
        g, ow_ref[...], (((1,), (1,)), ((), ())),
        preferred_element_type=jnp.float32,
    )


@jax.jit
def kernel(x, expert_w, gate_w, out_w):
    bsz, seql, _ = x.shape
    n_tok = bsz * seql
    xf = x.reshape(n_tok, INPUT_DIM)
    ew = (expert_w.astype(jnp.bfloat16)
          .transpose(1, 0, 2).reshape(INTER_DIM, GATE_NUM * INPUT_DIM))
    ow = out_w.astype(jnp.bfloat16)
    out = pl.pallas_call(
        _body,
        grid=(n_tok // TOK_BLK,),
        in_specs=[
            pl.BlockSpec((TOK_BLK, INPUT_DIM), lambda i: (i, 0)),
            pl.BlockSpec((INTER_DIM, GATE_NUM * INPUT_DIM),
                         lambda i: (0, 0)),
            pl.BlockSpec((GATE_NUM, INPUT_DIM), lambda i: (0, 0)),
            pl.BlockSpec((INPUT_DIM, INTER_DIM), lambda i: (0, 0)),
        ],
        out_specs=pl.BlockSpec((TOK_BLK, INPUT_DIM), lambda i: (i, 0)),
        out_shape=jax.ShapeDtypeStruct((n_tok, INPUT_DIM), jnp.float32),
    )(xf, ew, gate_w, ow)
    return out.reshape(bsz, seql, INPUT_DIM)


# speedup vs baseline: 1.1922x; 1.1922x over previous
import functools

import jax
import jax.numpy as jnp
from jax.experimental import pallas as pl


INPUT_DIM = 1024
INTER_DIM = 2048
GATE_NUM = 8
TOP_K = 2

TOK_BLK = 512


def _body(x_ref, ew_ref, gw_ref, ow_ref, o_ref):
    x = x_ref[...]
    xb = x.astype(jnp.bfloat16)
    scores = jax.lax.dot_general(
        xb, gw_ref[...].astype(jnp.bfloat16), (((1,), (1,)), ((), ())),
        preferred_element_type=jnp.float32,
    )
    idx = jax.lax.broadcasted_iota(jnp.int32, scores.shape, 1)
    m1 = jnp.max(scores, axis=1, keepdims=True)
    a1 = jnp.min(jnp.where(scores == m1, idx, GATE_NUM), axis=1, keepdims=True)
    scores2 = jnp.where(idx == a1, -jnp.inf, scores)
    m2 = jnp.max(scores2, axis=1, keepdims=True)
    a2 = jnp.min(jnp.where(scores2 == m2, idx, GATE_NUM), axis=1, keepdims=True)

    acc = jnp.zeros((x.shape[0], INTER_DIM), jnp.float32)
    for e in range(GATE_NUM):
        sel = ((a1 == e) | (a2 == e)).astype(jnp.float32)
        h = jax.lax.dot_general(
            xb, ew_ref[e], (((1,), (1,)), ((), ())),
            preferred_element_type=jnp.float32,
        )
        acc = acc + h * (m1 * sel)
    g = jnp.square(jnp.maximum(acc, 0.0)).astype(jnp.bfloat16)
    o_ref[...] = jax.lax.dot_general(
        g, ow_ref[...], (((1,), (1,)), ((), ())),
        preferred_element_type=jnp.float32,
    )


@jax.jit
def kernel(x, expert_w, gate_w, out_w):
    bsz, seql, _ = x.shape
    n_tok = bsz * seql
    xf = x.reshape(n_tok, INPUT_DIM)
    ew = expert_w.astype(jnp.bfloat16)
    ow = out_w.astype(jnp.bfloat16)
    out = pl.pallas_call(
        _body,
        grid=(n_tok // TOK_BLK,),
        in_specs=[
            pl.BlockSpec((TOK_BLK, INPUT_DIM), lambda i: (i, 0)),
            pl.BlockSpec((GATE_NUM, INTER_DIM, INPUT_DIM), lambda i: (0, 0, 0)),
            pl.BlockSpec((GATE_NUM, INPUT_DIM), lambda i: (0, 0)),
            pl.BlockSpec((INPUT_DIM, INTER_DIM), lambda i: (0, 0)),
        ],
        out_specs=pl.BlockSpec((TOK_BLK, INPUT_DIM), lambda i: (i, 0)),
        out_shape=jax.ShapeDtypeStruct((n_tok, INPUT_DIM), jnp.float32),
    )(xf, ew, gate_w, ow)
    return out.reshape(bsz, seql, INPUT_DIM)
